# Initial kernel scaffold; baseline (speedup 1.0000x reference)
import jax
import jax.numpy as jnp
from jax.experimental import pallas as pl


def kernel(edge_attr, emb0, emb1, emb2):
    raise NotImplementedError("write your pallas kernel here")


# speedup vs baseline: 9.5300x; 9.5300x over previous
import functools

import jax
import jax.numpy as jnp
from jax import lax
from jax.experimental import pallas as pl
from jax.experimental.pallas import tpu as pltpu
from jax.experimental.pallas import tpu_sc as plsc

E = 320000
D = 128
V = 20

NW = 32
CHUNK = 512
GROUP = 128
NGROUP = CHUNK // GROUP
NCHUNKS = E // CHUNK
CHUNKS_PER_W = -(-NCHUNKS // NW)


def _combos_body(e0_ref, e1_ref, e2_ref, out_ref):
    i0 = pl.program_id(0)
    t01 = e0_ref[i0, :][None, :] + e1_ref[...]
    blk = t01[:, None, :] + e2_ref[...][None, :, :]
    out_ref[...] = blk.reshape(V * V, D)


def _combos(emb0, emb1, emb2):
    return pl.pallas_call(
        _combos_body,
        grid=(V,),
        in_specs=[
            pl.BlockSpec((V, D), lambda i: (0, 0)),
            pl.BlockSpec((V, D), lambda i: (0, 0)),
            pl.BlockSpec((V, D), lambda i: (0, 0)),
        ],
        out_specs=pl.BlockSpec((V * V, D), lambda i: (i, 0)),
        out_shape=jax.ShapeDtypeStruct((V * V * V, D), jnp.float32),
    )(emb0, emb1, emb2)


@functools.partial(
    pl.kernel,
    mesh=plsc.VectorSubcoreMesh(core_axis_name="c", subcore_axis_name="s"),
    out_type=jax.ShapeDtypeStruct((E, D), jnp.float32),
    scratch_types=[
        pltpu.VMEM((CHUNK,), jnp.int32),
        pltpu.VMEM((CHUNK,), jnp.int32),
        pltpu.VMEM((CHUNK,), jnp.int32),
        pltpu.VMEM((CHUNK,), jnp.int32),
        pltpu.VMEM((CHUNK, D), jnp.float32),
        pltpu.SemaphoreType.DMA,
    ],
)
def _sc_gather(attr0_hbm, attr1_hbm, attr2_hbm, combos_hbm, out_hbm,
               a0_v, a1_v, a2_v, cidx_v, rows_v, sem):
    wid = lax.axis_index("s") * 2 + lax.axis_index("c")

    def chunk_body(i, carry):
        t = wid + i * NW
        @pl.when(t < NCHUNKS)
        def _():
            base = t * CHUNK
            pltpu.sync_copy(attr0_hbm.at[pl.ds(base, CHUNK)], a0_v)
            pltpu.sync_copy(attr1_hbm.at[pl.ds(base, CHUNK)], a1_v)
            pltpu.sync_copy(attr2_hbm.at[pl.ds(base, CHUNK)], a2_v)
            for j in range(CHUNK // 16):
                sl = pl.ds(j * 16, 16)
                c = a0_v[sl] * 400 + a1_v[sl] * 20 + a2_v[sl]
                cidx_v[sl] = c
            copies = []
            for g in range(NGROUP):
                copies.append(pltpu.async_copy(
                    combos_hbm.at[cidx_v.at[pl.ds(g * GROUP, GROUP)]],
                    rows_v.at[pl.ds(g * GROUP, GROUP)],
                    sem,
                ))
            for cp in copies:
                cp.wait()
            pltpu.sync_copy(rows_v, out_hbm.at[pl.ds(base, CHUNK)])
        return carry

    lax.fori_loop(0, CHUNKS_PER_W, chunk_body, 0)


def kernel(edge_attr, emb0, emb1, emb2):
    combos = _combos(emb0, emb1, emb2)
    attr_t = edge_attr.T
    return _sc_gather(attr_t[0], attr_t[1], attr_t[2], combos)


# speedup vs baseline: 11.9013x; 1.2488x over previous
import functools

import jax
import jax.numpy as jnp
from jax import lax
from jax.experimental import pallas as pl
from jax.experimental.pallas import tpu as pltpu
from jax.experimental.pallas import tpu_sc as plsc

E = 320000
D = 128
V = 20

NW = 32
PER_W = E // NW
GROUP = 80
NBUF = 5
NOUTER = PER_W // (GROUP * NBUF)
FUSE_IT = PER_W // 16


def _combos_body(e0_ref, e1_ref, e2_ref, out_ref):
    i0 = pl.program_id(0)
    t01 = e0_ref[i0, :][None, :] + e1_ref[...]
    blk = t01[:, None, :] + e2_ref[...][None, :, :]
    out_ref[...] = blk.reshape(V * V, D)


def _combos(emb0, emb1, emb2):
    return pl.pallas_call(
        _combos_body,
        grid=(V,),
        in_specs=[
            pl.BlockSpec((V, D), lambda i: (0, 0)),
            pl.BlockSpec((V, D), lambda i: (0, 0)),
            pl.BlockSpec((V, D), lambda i: (0, 0)),
        ],
        out_specs=pl.BlockSpec((V * V, D), lambda i: (i, 0)),
        out_shape=jax.ShapeDtypeStruct((V * V * V, D), jnp.float32),
    )(emb0, emb1, emb2)


@functools.partial(
    pl.kernel,
    mesh=plsc.VectorSubcoreMesh(core_axis_name="c", subcore_axis_name="s"),
    out_type=jax.ShapeDtypeStruct((E, D), jnp.float32),
    scratch_types=(
        [pltpu.VMEM((PER_W,), jnp.int32)] * 3
        + [pltpu.VMEM((PER_W,), jnp.int32)]
        + [pltpu.VMEM((GROUP, D), jnp.float32)] * NBUF
        + [pltpu.SemaphoreType.DMA] * (1 + 2 * NBUF)
    ),
)
def _sc_gather(attr0_hbm, attr1_hbm, attr2_hbm, combos_hbm, out_hbm,
               a0_v, a1_v, a2_v, cidx_v, *bufs_and_sems):
    rows = bufs_and_sems[:NBUF]
    isem = bufs_and_sems[NBUF]
    gsem = bufs_and_sems[NBUF + 1:2 * NBUF + 1]
    ssem = bufs_and_sems[2 * NBUF + 1:]
    wid = lax.axis_index("s") * 2 + lax.axis_index("c")
    base = wid * PER_W

    cps = [pltpu.async_copy(a.at[pl.ds(base, PER_W)], v, isem)
           for a, v in ((attr0_hbm, a0_v), (attr1_hbm, a1_v),
                        (attr2_hbm, a2_v))]
    for cp in cps:
        cp.wait()

    def fuse_body(j, carry):
        sl = pl.ds(pl.multiple_of(j * 16, 16), 16)
        cidx_v[sl] = a0_v[sl] * 400 + a1_v[sl] * 20 + a2_v[sl]
        return carry

    lax.fori_loop(0, FUSE_IT, fuse_body, 0)

    def outer_body(o, carry):
        goff = pl.multiple_of(o * (GROUP * NBUF), GROUP * NBUF)
        gcps = []
        for b in range(NBUF):
            @pl.when(o > 0)
            def _(b=b):
                pltpu.make_async_copy(
                    rows[b], out_hbm.at[pl.ds(0, GROUP)], ssem[b]).wait()
            cidx_sl = cidx_v.at[pl.ds(goff + b * GROUP, GROUP)]
            gcps.append(pltpu.async_copy(
                combos_hbm.at[cidx_sl], rows[b], gsem[b]))
        for b in range(NBUF):
            gcps[b].wait()
            pltpu.async_copy(
                rows[b], out_hbm.at[pl.ds(base + goff + b * GROUP, GROUP)],
                ssem[b])
        return carry

    lax.fori_loop(0, NOUTER, outer_body, 0)
    for b in range(NBUF):
        pltpu.make_async_copy(
            rows[b], out_hbm.at[pl.ds(0, GROUP)], ssem[b]).wait()


def kernel(edge_attr, emb0, emb1, emb2):
    combos = _combos(emb0, emb1, emb2)
    attr_t = edge_attr.T
    return _sc_gather(attr_t[0], attr_t[1], attr_t[2], combos)
